# Initial kernel scaffold; baseline (speedup 1.0000x reference)
import jax
import jax.numpy as jnp
from jax.experimental import pallas as pl


def kernel(x, edge_index, edge_attr, batch, params):
    raise NotImplementedError("write your pallas kernel here")


# speedup vs baseline: 2.3196x; 2.3196x over previous
import functools

import jax
import jax.numpy as jnp
from jax import lax
from jax.experimental import pallas as pl
from jax.experimental.pallas import tpu as pltpu
from jax.experimental.pallas import tpu_sc as plsc

F32 = jnp.float32
N_NODES = 4096
N_EDGES = 8192
N_GRAPHS = 256
D_NODE = 11

NC, NS = 2, 16
NW = NC * NS

@functools.lru_cache(maxsize=None)
def _mesh():
    return plsc.VectorSubcoreMesh(
        core_axis_name="c", subcore_axis_name="s", num_cores=NC, num_subcores=NS
    )


def _leaky(v):
    return jnp.where(v >= 0, v, 0.01 * v)


@functools.lru_cache(maxsize=None)
def _gather_call(d):
    def body(table, idx2d, out, idx_v, rows_v, sem):
        c = lax.axis_index("c")
        s = lax.axis_index("s")
        wid = c * NS + s
        for j in range(2):
            pltpu.sync_copy(idx2d.at[wid * 2 + j], idx_v.at[j])
            pltpu.async_copy(table.at[idx_v.at[j]], rows_v, sem).wait()
            pltpu.sync_copy(rows_v, out.at[pl.ds(wid * 256 + j * 128, 128)])

    return pl.kernel(
        body,
        out_type=jax.ShapeDtypeStruct((N_EDGES, d), F32),
        mesh=_mesh(),
        scratch_types=[
            pltpu.VMEM((2, 128), jnp.int32),
            pltpu.VMEM((128, d), F32),
            pltpu.SemaphoreType.DMA,
        ],
    )


def _seg_sum_call(data, idx3d, n_out):
    rows, d = data.shape
    ebk = 512
    nbk = min(1024, n_out)
    grid = (n_out // nbk, rows // ebk)

    def body(idx_ref, data_ref, out_ref):
        n = pl.program_id(0)
        e = pl.program_id(1)
        row_ids = jax.lax.broadcasted_iota(jnp.int32, (nbk, ebk), 0) + n * nbk
        m = (row_ids == idx_ref[0]).astype(F32)
        contrib = jnp.dot(m, data_ref[...], preferred_element_type=F32)

        @pl.when(e == 0)
        def _():
            out_ref[...] = contrib

        @pl.when(e > 0)
        def _():
            out_ref[...] += contrib

    return pl.pallas_call(
        body,
        grid=grid,
        in_specs=[
            pl.BlockSpec((1, 1, ebk), lambda n, e: (e, 0, 0)),
            pl.BlockSpec((ebk, d), lambda n, e: (e, 0)),
        ],
        out_specs=pl.BlockSpec((nbk, d), lambda n, e: (n, 0)),
        out_shape=jax.ShapeDtypeStruct((n_out, d), F32),
    )(idx3d, data)


EB = 256
IC = 16


def _msg_call(ea, xs, w1, b1, w2r, b2r, hid, d_in_pad, d_out, append_ones, d_store):
    n_i = d_in_pad // IC
    kw = IC * hid
    d_xs = xs.shape[-1]

    def body(ea_ref, xs_ref, w1_ref, b1_ref, w2_ref, b2_ref, out_ref):
        h = _leaky(
            jnp.dot(ea_ref[...], w1_ref[...], preferred_element_type=F32)
            + b1_ref[...]
        )
        xs_full = xs_ref[...][:, :d_in_pad]
        acc = jnp.dot(xs_full, b2_ref[...], preferred_element_type=F32)
        for i in range(n_i):
            xs_c = xs_full[:, i * IC : (i + 1) * IC]
            kron = (xs_c[:, :, None] * h[:, None, :]).reshape(EB, kw)
            acc = acc + jnp.dot(
                kron, w2_ref[i * kw : (i + 1) * kw, :], preferred_element_type=F32
            )
        pieces = [acc]
        if append_ones:
            pieces.append(jnp.ones((EB, 16), F32))
        cur = d_out + (16 if append_ones else 0)
        if d_store > cur:
            pieces.append(jnp.zeros((EB, d_store - cur), F32))
        out_ref[...] = pieces[0] if len(pieces) == 1 else jnp.concatenate(pieces, 1)

    return pl.pallas_call(
        body,
        grid=(N_EDGES // EB,),
        in_specs=[
            pl.BlockSpec((EB, 4), lambda e: (e, 0)),
            pl.BlockSpec((EB, d_xs), lambda e: (e, 0)),
            pl.BlockSpec((4, hid), lambda e: (0, 0)),
            pl.BlockSpec((1, hid), lambda e: (0, 0)),
            pl.BlockSpec((d_in_pad * hid, d_out), lambda e: (0, 0)),
            pl.BlockSpec((d_in_pad, d_out), lambda e: (0, 0)),
        ],
        out_specs=pl.BlockSpec((EB, d_store), lambda e: (e, 0)),
        out_shape=jax.ShapeDtypeStruct((N_EDGES, d_store), F32),
    )(ea, xs, w1, b1, w2r, b2r)


NB = 256


def _post_call(part, cnt16, xprev, rootp, bias, x0p, d_out, d_prev_pad, emit_cnt,
               append_ones, d_store):
    d_part = part.shape[-1]
    d_prev_store = xprev.shape[-1]

    def body(*refs):
        if emit_cnt:
            part_ref, xprev_ref, root_ref, bias_ref, x0_ref, out_ref, cnt_ref = refs
            ps = part_ref[...]
            cnt = ps[:, d_out : d_out + 16]
            cnt_ref[...] = cnt
        else:
            part_ref, cnt_in_ref, xprev_ref, root_ref, bias_ref, x0_ref, out_ref = refs
            ps = part_ref[...]
            cnt = cnt_in_ref[...]
        cdiv = jnp.maximum(cnt[:, 0:1], 1.0)
        agg = ps[:, 0:d_out] / cdiv
        conv = _leaky(
            agg
            + jnp.dot(
                xprev_ref[...][:, :d_prev_pad],
                root_ref[...],
                preferred_element_type=F32,
            )
            + bias_ref[...]
        )
        pieces = [conv, x0_ref[...]]
        cur = d_out + 16
        if append_ones:
            pieces.append(jnp.ones((NB, 16), F32))
            cur += 16
        if d_store > cur:
            pieces.append(jnp.zeros((NB, d_store - cur), F32))
        out_ref[...] = jnp.concatenate(pieces, axis=1)

    in_specs = [pl.BlockSpec((NB, d_part), lambda n: (n, 0))]
    args = [part]
    if not emit_cnt:
        in_specs.append(pl.BlockSpec((NB, 16), lambda n: (n, 0)))
        args.append(cnt16)
    in_specs += [
        pl.BlockSpec((NB, d_prev_store), lambda n: (n, 0)),
        pl.BlockSpec((d_prev_pad, d_out), lambda n: (0, 0)),
        pl.BlockSpec((1, d_out), lambda n: (0, 0)),
        pl.BlockSpec((NB, 16), lambda n: (n, 0)),
    ]
    args += [xprev, rootp, bias, x0p]
    out_shapes = jax.ShapeDtypeStruct((N_NODES, d_store), F32)
    out_specs = pl.BlockSpec((NB, d_store), lambda n: (n, 0))
    if emit_cnt:
        out_shapes = (out_shapes, jax.ShapeDtypeStruct((N_NODES, 16), F32))
        out_specs = (out_specs, pl.BlockSpec((NB, 16), lambda n: (n, 0)))
    return pl.pallas_call(
        body,
        grid=(N_NODES // NB,),
        in_specs=in_specs,
        out_specs=out_specs,
        out_shape=out_shapes,
    )(*args)


def _head_call(pp, fc1wp, fc1b, fc2w, fc2b, fc3w, fc3b):
    def body(pp_ref, w1_ref, b1_ref, w2_ref, b2_ref, w3_ref, b3_ref, out_ref):
        s = pp_ref[...]
        cnt = jnp.maximum(s[:, 272:273], 1.0)
        pf = s[:, 0:272] / cnt
        h1 = _leaky(jnp.dot(pf, w1_ref[...], preferred_element_type=F32) + b1_ref[...])
        h2 = _leaky(jnp.dot(h1, w2_ref[...], preferred_element_type=F32) + b2_ref[...])
        out_ref[...] = (
            jnp.dot(h2, w3_ref[...], preferred_element_type=F32) + b3_ref[...]
        )

    return pl.pallas_call(
        body,
        out_shape=jax.ShapeDtypeStruct((N_GRAPHS, 1), F32),
    )(pp, fc1wp, fc1b, fc2w, fc2b, fc3w, fc3b)


def _prep_layer(p, d_in, d_in_pad, hid, d_out):
    w2r = p["w2"].reshape(hid, d_in, d_out).transpose(1, 0, 2)
    w2r = jnp.pad(w2r, ((0, d_in_pad - d_in), (0, 0), (0, 0)))
    w2r = w2r.reshape(d_in_pad * hid, d_out)
    b2r = jnp.pad(p["b2"].reshape(d_in, d_out), ((0, d_in_pad - d_in), (0, 0)))
    rootp = jnp.pad(p["root"], ((0, d_in_pad - d_in), (0, 0)))
    return (
        p["w1"],
        p["b1"].reshape(1, hid),
        w2r,
        b2r,
        rootp,
        p["bias"].reshape(1, d_out),
    )


def kernel(x, edge_index, edge_attr, batch, params):
    idx_src = edge_index[0].reshape(64, 128)
    idx_dst = edge_index[1].reshape(N_EDGES // 512, 1, 512)
    idx_batch = batch.reshape(N_NODES // 512, 1, 512)
    x0p = jnp.pad(x, ((0, 0), (0, 16 - D_NODE)))
    x0t = jnp.pad(x, ((0, 0), (0, 128 - D_NODE)))

    w1_1, b1_1, w2r_1, b2r_1, root_1, bias_1 = _prep_layer(
        params["conv1"], D_NODE, 16, 128, 128
    )
    w1_2, b1_2, w2r_2, b2r_2, root_2, bias_2 = _prep_layer(
        params["conv2"], 128 + D_NODE, 144, 256, 128
    )
    w1_3, b1_3, w2r_3, b2r_3, root_3, bias_3 = _prep_layer(
        params["conv3"], 128 + D_NODE, 144, 256, 256
    )
    fc1wp = jnp.pad(params["fc1"]["w"], ((0, 272 - 267), (0, 0)))
    fc1b = params["fc1"]["b"].reshape(1, -1)
    fc2b = params["fc2"]["b"].reshape(1, -1)
    fc3b = params["fc3"]["b"].reshape(1, -1)

    xs1 = _gather_call(128)(x0t, idx_src)
    msg1 = _msg_call(edge_attr, xs1, w1_1, b1_1, w2r_1, b2r_1, 128, 16, 128, True, 256)
    part1 = _seg_sum_call(msg1, idx_dst, N_NODES)
    d1p, cnt16 = _post_call(
        part1, None, x0t, root_1, bias_1, x0p, 128, 16, True, False, 256
    )
    xs2 = _gather_call(256)(d1p, idx_src)
    msg2 = _msg_call(edge_attr, xs2, w1_2, b1_2, w2r_2, b2r_2, 256, 144, 128, False, 128)
    part2 = _seg_sum_call(msg2, idx_dst, N_NODES)
    d2p = _post_call(part2, cnt16, d1p, root_2, bias_2, x0p, 128, 144, False, False, 256)
    xs3 = _gather_call(256)(d2p, idx_src)
    msg3 = _msg_call(edge_attr, xs3, w1_3, b1_3, w2r_3, b2r_3, 256, 144, 256, False, 256)
    part3 = _seg_sum_call(msg3, idx_dst, N_NODES)
    d3p = _post_call(part3, cnt16, d2p, root_3, bias_3, x0p, 256, 144, False, True, 384)
    pp = _seg_sum_call(d3p, idx_batch, N_GRAPHS)
    return _head_call(
        pp, fc1wp, fc1b, params["fc2"]["w"], fc2b, params["fc3"]["w"], fc3b
    )
